# baseline (speedup 1.0000x reference)
import functools

import jax
import jax.numpy as jnp
import numpy as np
from jax import lax
from jax.experimental import pallas as pl
from jax.experimental.pallas import tpu as pltpu
from jax.experimental.pallas import tpu_sc as plsc

N = 10000
E = 160000
D = 128
SH = 4
NB = 8
MID = D * SH
N_ATTR = 10
AVG = 16.0

NW = 32
OP = 128
NCHUNK = 2
OPW = 20
E_CH = NW * OPW * OP
E_PAD = NCHUNK * E_CH
N_PAD = 10240
ROWS_PER_TILE = N_PAD // 16

EDGE_BLOCK = 2048
NODE_BLOCK = 1000
OUT_BLOCK = 1024


def _mesh():
    return plsc.VectorSubcoreMesh(core_axis_name="c", subcore_axis_name="s")


def _wnf_body(nf_ref, w_ref, o_ref):
    o_ref[...] = jnp.dot(nf_ref[...], w_ref[...],
                         preferred_element_type=jnp.float32)


def _wnf_call(node_feats, w_node):
    return pl.pallas_call(
        _wnf_body,
        grid=(N // NODE_BLOCK,),
        in_specs=[
            pl.BlockSpec((NODE_BLOCK, D), lambda i: (i, 0)),
            pl.BlockSpec((D, D), lambda i: (0, 0)),
        ],
        out_specs=pl.BlockSpec((NODE_BLOCK, D), lambda i: (i, 0)),
        out_shape=jax.ShapeDtypeStruct((N, D), jnp.float32),
    )(node_feats, w_node)


def _sc_gather_call(table, idx3d):
    @functools.partial(
        pl.kernel,
        out_type=jax.ShapeDtypeStruct((E_CH, D // 2), jnp.int32),
        mesh=_mesh(),
        compiler_params=pltpu.CompilerParams(use_tc_tiling_on_sc=False),
        scratch_types=[
            pltpu.VMEM((OPW, OP), jnp.int32),
            pltpu.VMEM((OP, D // 2), jnp.int32),
            pltpu.VMEM((OP, D // 2), jnp.int32),
            pltpu.SemaphoreType.DMA,
            pltpu.SemaphoreType.DMA,
        ],
    )
    def k(table_hbm, idx_hbm, out_hbm, idx_v, rows_a, rows_b, gsem, ssem):
        wid = lax.axis_index("s") * 2 + lax.axis_index("c")
        row0 = wid * OPW
        pltpu.sync_copy(idx_hbm.at[wid], idx_v)

        def body(p, carry):
            t0 = 2 * p
            t1 = t0 + 1
            ga = pltpu.async_copy(table_hbm.at[idx_v.at[t0]], rows_a, gsem)
            gb = pltpu.async_copy(table_hbm.at[idx_v.at[t1]], rows_b, gsem)
            ga.wait()
            sa = pltpu.async_copy(rows_a,
                                  out_hbm.at[pl.ds((row0 + t0) * OP, OP)],
                                  ssem)
            gb.wait()
            sb = pltpu.async_copy(rows_b,
                                  out_hbm.at[pl.ds((row0 + t1) * OP, OP)],
                                  ssem)
            sa.wait()
            sb.wait()
            return carry

        lax.fori_loop(0, OPW // 2, body, 0)

    return k(table, idx3d)


def _edge_body(ef_ref, ea_ref, hs_ref, w1_ref, w2_ref, w3_ref, w4_ref,
               wmsg_ref, o_ref):
    x = ef_ref[...]
    h = jax.nn.silu(jnp.dot(x, w1_ref[...], preferred_element_type=jnp.float32))
    h = jax.nn.silu(jnp.dot(h, w2_ref[...], preferred_element_type=jnp.float32))
    h = jax.nn.silu(jnp.dot(h, w3_ref[...], preferred_element_type=jnp.float32))
    w = jnp.dot(h, w4_ref[...], preferred_element_type=jnp.float32)
    hs = hs_ref[...].astype(jnp.float32)
    ea = ea_ref[...]
    wmsg = wmsg_ref[...]
    acc = jnp.zeros((EDGE_BLOCK, D), jnp.float32)
    for m in range(SH):
        t = hs * w[:, m * D:(m + 1) * D] * ea[:, m:m + 1]
        acc = acc + jnp.dot(t, wmsg[m * D:(m + 1) * D, :],
                            preferred_element_type=jnp.float32)
    o_ref[...] = acc


def _edge_call(ef, ea, hs_ch, w1, w2, w3, w4, wmsg, chunk):
    blk0 = chunk * (E_CH // EDGE_BLOCK)
    return pl.pallas_call(
        _edge_body,
        grid=(E_CH // EDGE_BLOCK,),
        in_specs=[
            pl.BlockSpec((EDGE_BLOCK, NB), lambda i: (i + blk0, 0)),
            pl.BlockSpec((EDGE_BLOCK, SH), lambda i: (i + blk0, 0)),
            pl.BlockSpec((EDGE_BLOCK, D), lambda i: (i, 0)),
            pl.BlockSpec((NB, 64), lambda i: (0, 0)),
            pl.BlockSpec((64, 64), lambda i: (0, 0)),
            pl.BlockSpec((64, 64), lambda i: (0, 0)),
            pl.BlockSpec((64, MID), lambda i: (0, 0)),
            pl.BlockSpec((MID, D), lambda i: (0, 0)),
        ],
        out_specs=pl.BlockSpec((EDGE_BLOCK, D), lambda i: (i, 0)),
        out_shape=jax.ShapeDtypeStruct((E_CH, D), jnp.float32),
    )(ef, ea, hs_ch, w1, w2, w3, w4, wmsg)


def _sc_scatter_call(msg_ch, ridx3d, init):
    @functools.partial(
        pl.kernel,
        out_type=jax.ShapeDtypeStruct((2, N_PAD, D), jnp.float32),
        mesh=_mesh(),
        scratch_types=[
            pltpu.VMEM((OPW, OP), jnp.int32),
            pltpu.VMEM((OP, D), jnp.float32),
            pltpu.VMEM((OP, D), jnp.float32),
            pltpu.VMEM_SHARED((N_PAD, D), jnp.float32),
            pltpu.SemaphoreType.DMA,
            pltpu.SemaphoreType.DMA,
        ],
    )
    def k(msg_hbm, idx_hbm, init_hbm, out_hbm, idx_v, rows_a, rows_b, acc_sh,
          lsem, scsem):
        c = lax.axis_index("c")
        s = lax.axis_index("s")
        wid = s * 2 + c
        r0 = s * ROWS_PER_TILE
        pltpu.sync_copy(init_hbm.at[c, pl.ds(r0, ROWS_PER_TILE)],
                        acc_sh.at[pl.ds(r0, ROWS_PER_TILE)])
        plsc.subcore_barrier()
        row0 = wid * OPW
        pltpu.sync_copy(idx_hbm.at[wid], idx_v)

        def body(p, carry):
            t0 = 2 * p
            t1 = t0 + 1
            la = pltpu.async_copy(msg_hbm.at[pl.ds((row0 + t0) * OP, OP)],
                                  rows_a, lsem)
            lb = pltpu.async_copy(msg_hbm.at[pl.ds((row0 + t1) * OP, OP)],
                                  rows_b, lsem)
            la.wait()
            sa = pltpu.async_copy(rows_a, acc_sh.at[idx_v.at[t0]], scsem,
                                  add=True)
            lb.wait()
            sb = pltpu.async_copy(rows_b, acc_sh.at[idx_v.at[t1]], scsem,
                                  add=True)
            sa.wait()
            sb.wait()
            return carry

        lax.fori_loop(0, OPW // 2, body, 0)
        plsc.subcore_barrier()
        pltpu.sync_copy(acc_sh.at[pl.ds(r0, ROWS_PER_TILE)],
                        out_hbm.at[c, pl.ds(r0, ROWS_PER_TILE)])

    return k(msg_ch, ridx3d, init)


def _out_body(p_ref, na_ref, wout_ref, o_ref):
    wm = p_ref[0] + p_ref[1]
    na = na_ref[...]
    wout = wout_ref[...]
    acc = jnp.zeros((OUT_BLOCK, D), jnp.float32)
    for a in range(N_ATTR):
        acc = acc + jnp.dot(wm * na[:, a:a + 1], wout[a * D:(a + 1) * D, :],
                            preferred_element_type=jnp.float32)
    o_ref[...] = acc


def _out_call(parts, node_attrs_p, wout):
    return pl.pallas_call(
        _out_body,
        grid=(N_PAD // OUT_BLOCK,),
        in_specs=[
            pl.BlockSpec((2, OUT_BLOCK, D), lambda i: (0, i, 0)),
            pl.BlockSpec((OUT_BLOCK, N_ATTR), lambda i: (i, 0)),
            pl.BlockSpec((N_ATTR * D, D), lambda i: (0, 0)),
        ],
        out_specs=pl.BlockSpec((OUT_BLOCK, D), lambda i: (i, 0)),
        out_shape=jax.ShapeDtypeStruct((N_PAD, D), jnp.float32),
    )(parts, node_attrs_p, wout)


def kernel(node_attrs, node_feats, edge_attrs, edge_feats, edge_index,
           W_node, Wm1, Wm2, Wm3, Wm4, W_msg, W_out):
    sender = edge_index[0].astype(jnp.int32)
    receiver = edge_index[1].astype(jnp.int32)

    pad = E_PAD - E
    sender_c = jnp.pad(sender, (0, pad)).reshape(NCHUNK, NW, OPW, OP)
    receiver_c = jnp.pad(receiver, (0, pad)).reshape(NCHUNK, NW, OPW, OP)
    ef_p = jnp.pad(edge_feats, ((0, pad), (0, 0)))
    ea_p = jnp.pad(edge_attrs, ((0, pad), (0, 0)))

    w1 = Wm1 / np.sqrt(NB)
    w2 = Wm2 / np.sqrt(64.0)
    w3 = Wm3 / np.sqrt(64.0)
    w4 = (Wm4 / np.sqrt(64.0)).reshape(64, D, SH).transpose(0, 2, 1)
    w4 = w4.reshape(64, MID)
    wmsg = (W_msg / AVG).reshape(D, SH, D).transpose(1, 0, 2).reshape(MID, D)
    wout = W_out.transpose(1, 0, 2).reshape(N_ATTR * D, D)

    wnf = _wnf_call(node_feats, W_node)
    wnf_packed = lax.bitcast_convert_type(
        wnf.astype(jnp.bfloat16).reshape(N, D // 2, 2), jnp.int32)
    parts = jnp.zeros((2, N_PAD, D), jnp.float32)
    for h in range(NCHUNK):
        hs_i = _sc_gather_call(wnf_packed, sender_c[h])
        hs = lax.bitcast_convert_type(hs_i, jnp.bfloat16).reshape(E_CH, D)
        msg = _edge_call(ef_p, ea_p, hs, w1, w2, w3, w4, wmsg, h)
        parts = _sc_scatter_call(msg, receiver_c[h], parts)
    na_p = jnp.pad(node_attrs, ((0, N_PAD - N), (0, 0)))
    return _out_call(parts, na_p, wout)[:N]


# speedup vs baseline: 1.5664x; 1.5664x over previous
import functools

import jax
import jax.numpy as jnp
import numpy as np
from jax import lax
from jax.experimental import pallas as pl
from jax.experimental.pallas import tpu as pltpu
from jax.experimental.pallas import tpu_sc as plsc

N = 10000
E = 160000
D = 128
SH = 4
NB = 8
MID = D * SH
N_ATTR = 10
AVG = 16.0

NW = 32
OP = 128
NCHUNK = 2
OPW = 20
E_CH = NW * OPW * OP
E_PAD = NCHUNK * E_CH
N_PAD = 10240
ROWS_PER_TILE = N_PAD // 16

EDGE_BLOCK = 2048
NODE_BLOCK = 1000
OUT_BLOCK = 1024


def _mesh():
    return plsc.VectorSubcoreMesh(core_axis_name="c", subcore_axis_name="s")


def _wnf_body(nf_ref, w_ref, o_ref):
    o_ref[...] = jnp.dot(nf_ref[...], w_ref[...],
                         preferred_element_type=jnp.float32)


def _wnf_call(node_feats, w_node):
    return pl.pallas_call(
        _wnf_body,
        grid=(N // NODE_BLOCK,),
        in_specs=[
            pl.BlockSpec((NODE_BLOCK, D), lambda i: (i, 0)),
            pl.BlockSpec((D, D), lambda i: (0, 0)),
        ],
        out_specs=pl.BlockSpec((NODE_BLOCK, D), lambda i: (i, 0)),
        out_shape=jax.ShapeDtypeStruct((N, D), jnp.float32),
    )(node_feats, w_node)


def _sc_gather_call(table, idx3d):
    @functools.partial(
        pl.kernel,
        out_type=jax.ShapeDtypeStruct((E_CH, D), jnp.float32),
        mesh=_mesh(),
        scratch_types=[
            pltpu.VMEM((OPW, OP), jnp.int32),
            [pltpu.VMEM((OP, D), jnp.float32) for _ in range(4)],
            pltpu.SemaphoreType.DMA,
            pltpu.SemaphoreType.DMA,
        ],
    )
    def k(table_hbm, idx_hbm, out_hbm, idx_v, rows, gsem, ssem):
        wid = lax.axis_index("s") * 2 + lax.axis_index("c")
        row0 = wid * OPW
        pltpu.sync_copy(idx_hbm.at[wid], idx_v)

        def body(p, carry):
            base = 4 * p
            gs = [pltpu.async_copy(table_hbm.at[idx_v.at[base + j]], rows[j],
                                   gsem) for j in range(4)]
            sts = []
            for j in range(4):
                gs[j].wait()
                sts.append(pltpu.async_copy(
                    rows[j], out_hbm.at[pl.ds((row0 + base + j) * OP, OP)],
                    ssem))
            for st in sts:
                st.wait()
            return carry

        lax.fori_loop(0, OPW // 4, body, 0)

    return k(table, idx3d)


def _edge_body(ef_ref, ea_ref, hs_ref, w1_ref, w2_ref, w3_ref, w4_ref,
               wmsg_ref, o_ref):
    x = ef_ref[...]
    h = jax.nn.silu(jnp.dot(x, w1_ref[...], preferred_element_type=jnp.float32))
    h = jax.nn.silu(jnp.dot(h, w2_ref[...], preferred_element_type=jnp.float32))
    h = jax.nn.silu(jnp.dot(h, w3_ref[...], preferred_element_type=jnp.float32))
    w = jnp.dot(h, w4_ref[...], preferred_element_type=jnp.float32)
    hs = hs_ref[...]
    ea = ea_ref[...]
    wmsg = wmsg_ref[...]
    acc = jnp.zeros((EDGE_BLOCK, D), jnp.float32)
    for m in range(SH):
        t = hs * w[:, m * D:(m + 1) * D] * ea[:, m:m + 1]
        acc = acc + jnp.dot(t, wmsg[m * D:(m + 1) * D, :],
                            preferred_element_type=jnp.float32)
    o_ref[...] = acc


def _edge_call(ef, ea, hs_ch, w1, w2, w3, w4, wmsg, chunk):
    blk0 = chunk * (E_CH // EDGE_BLOCK)
    return pl.pallas_call(
        _edge_body,
        grid=(E_CH // EDGE_BLOCK,),
        in_specs=[
            pl.BlockSpec((EDGE_BLOCK, NB), lambda i: (i + blk0, 0)),
            pl.BlockSpec((EDGE_BLOCK, SH), lambda i: (i + blk0, 0)),
            pl.BlockSpec((EDGE_BLOCK, D), lambda i: (i, 0)),
            pl.BlockSpec((NB, 64), lambda i: (0, 0)),
            pl.BlockSpec((64, 64), lambda i: (0, 0)),
            pl.BlockSpec((64, 64), lambda i: (0, 0)),
            pl.BlockSpec((64, MID), lambda i: (0, 0)),
            pl.BlockSpec((MID, D), lambda i: (0, 0)),
        ],
        out_specs=pl.BlockSpec((EDGE_BLOCK, D), lambda i: (i, 0)),
        out_shape=jax.ShapeDtypeStruct((E_CH, D), jnp.float32),
    )(ef, ea, hs_ch, w1, w2, w3, w4, wmsg)


def _sc_scatter_call(msg_ch, ridx3d, init):
    @functools.partial(
        pl.kernel,
        out_type=jax.ShapeDtypeStruct((2, N_PAD, D), jnp.float32),
        mesh=_mesh(),
        scratch_types=[
            pltpu.VMEM((OPW, OP), jnp.int32),
            [pltpu.VMEM((OP, D), jnp.float32) for _ in range(2)],
            pltpu.VMEM_SHARED((N_PAD, D), jnp.float32),
            pltpu.SemaphoreType.DMA,
            pltpu.SemaphoreType.DMA,
        ],
    )
    def k(msg_hbm, idx_hbm, init_hbm, out_hbm, idx_v, rows, acc_sh,
          lsem, scsem):
        c = lax.axis_index("c")
        s = lax.axis_index("s")
        wid = s * 2 + c
        r0 = s * ROWS_PER_TILE
        pltpu.sync_copy(init_hbm.at[c, pl.ds(r0, ROWS_PER_TILE)],
                        acc_sh.at[pl.ds(r0, ROWS_PER_TILE)])
        plsc.subcore_barrier()
        row0 = wid * OPW
        pltpu.sync_copy(idx_hbm.at[wid], idx_v)

        def body(p, carry):
            base = 2 * p
            ls = [pltpu.async_copy(msg_hbm.at[pl.ds((row0 + base + j) * OP,
                                                    OP)], rows[j], lsem)
                  for j in range(2)]
            sts = []
            for j in range(2):
                ls[j].wait()
                sts.append(pltpu.async_copy(rows[j],
                                            acc_sh.at[idx_v.at[base + j]],
                                            scsem, add=True))
            for st in sts:
                st.wait()
            return carry

        lax.fori_loop(0, OPW // 2, body, 0)
        plsc.subcore_barrier()
        pltpu.sync_copy(acc_sh.at[pl.ds(r0, ROWS_PER_TILE)],
                        out_hbm.at[c, pl.ds(r0, ROWS_PER_TILE)])

    return k(msg_ch, ridx3d, init)


def _out_body(p_ref, na_ref, wout_ref, o_ref):
    wm = p_ref[0] + p_ref[1]
    na = na_ref[...]
    wout = wout_ref[...]
    acc = jnp.zeros((OUT_BLOCK, D), jnp.float32)
    for a in range(N_ATTR):
        acc = acc + jnp.dot(wm * na[:, a:a + 1], wout[a * D:(a + 1) * D, :],
                            preferred_element_type=jnp.float32)
    o_ref[...] = acc


def _out_call(parts, node_attrs_p, wout):
    return pl.pallas_call(
        _out_body,
        grid=(N_PAD // OUT_BLOCK,),
        in_specs=[
            pl.BlockSpec((2, OUT_BLOCK, D), lambda i: (0, i, 0)),
            pl.BlockSpec((OUT_BLOCK, N_ATTR), lambda i: (i, 0)),
            pl.BlockSpec((N_ATTR * D, D), lambda i: (0, 0)),
        ],
        out_specs=pl.BlockSpec((OUT_BLOCK, D), lambda i: (i, 0)),
        out_shape=jax.ShapeDtypeStruct((N_PAD, D), jnp.float32),
    )(parts, node_attrs_p, wout)


def kernel(node_attrs, node_feats, edge_attrs, edge_feats, edge_index,
           W_node, Wm1, Wm2, Wm3, Wm4, W_msg, W_out):
    sender = edge_index[0].astype(jnp.int32)
    receiver = edge_index[1].astype(jnp.int32)

    pad = E_PAD - E
    sender_c = jnp.pad(sender, (0, pad)).reshape(NCHUNK, NW, OPW, OP)
    receiver_c = jnp.pad(receiver, (0, pad)).reshape(NCHUNK, NW, OPW, OP)
    ef_p = jnp.pad(edge_feats, ((0, pad), (0, 0)))
    ea_p = jnp.pad(edge_attrs, ((0, pad), (0, 0)))

    w1 = Wm1 / np.sqrt(NB)
    w2 = Wm2 / np.sqrt(64.0)
    w3 = Wm3 / np.sqrt(64.0)
    w4 = (Wm4 / np.sqrt(64.0)).reshape(64, D, SH).transpose(0, 2, 1)
    w4 = w4.reshape(64, MID)
    wmsg = (W_msg / AVG).reshape(D, SH, D).transpose(1, 0, 2).reshape(MID, D)
    wout = W_out.transpose(1, 0, 2).reshape(N_ATTR * D, D)

    wnf = _wnf_call(node_feats, W_node)
    parts = jnp.zeros((2, N_PAD, D), jnp.float32)
    for h in range(NCHUNK):
        hs = _sc_gather_call(wnf, sender_c[h])
        msg = _edge_call(ef_p, ea_p, hs, w1, w2, w3, w4, wmsg, h)
        parts = _sc_scatter_call(msg, receiver_c[h], parts)
    na_p = jnp.pad(node_attrs, ((0, N_PAD - N), (0, 0)))
    return _out_call(parts, na_p, wout)[:N]
